# baseline (speedup 1.0000x reference)
import functools

import jax
import jax.numpy as jnp
from jax.experimental import pallas as pl
from jax.experimental.pallas import tpu as pltpu

NF = 384
K = 16


def _gather_body(idx_ref, *refs, C):
    x_refs = refs[:K]
    o_ref = refs[K]
    i = pl.program_id(0)
    for j in range(K):
        v = idx_ref[i * K + j]

        @pl.when(v < C)
        def _copy(j=j):
            o_ref[:, j, :, :] = x_refs[j][:, 0, :, :]

        @pl.when(v >= C)
        def _zero(j=j):
            o_ref[:, j, :, :] = jnp.zeros_like(o_ref[:, j, :, :])


def kernel(x, indices):
    B, C, H, W = x.shape

    def make_in_spec(j):
        return pl.BlockSpec(
            (B, 1, H, W),
            lambda i, idx_ref: (0, jnp.minimum(idx_ref[i * K + j], C - 1), 0, 0),
        )

    grid_spec = pltpu.PrefetchScalarGridSpec(
        num_scalar_prefetch=1,
        grid=(NF // K,),
        in_specs=[make_in_spec(j) for j in range(K)],
        out_specs=pl.BlockSpec((B, K, H, W), lambda i, idx_ref: (0, i, 0, 0)),
    )
    return pl.pallas_call(
        functools.partial(_gather_body, C=C),
        grid_spec=grid_spec,
        out_shape=jax.ShapeDtypeStruct((B, NF, H, W), x.dtype),
    )(indices, *([x] * K))


# speedup vs baseline: 1.0726x; 1.0726x over previous
import functools

import jax
import jax.numpy as jnp
from jax import lax
from jax.experimental import pallas as pl
from jax.experimental.pallas import tpu as pltpu
from jax.experimental.pallas import tpu_sc as plsc

NF = 384
G = 16


def kernel(x, indices):
    B, C, H, W = x.shape
    NP = B * NF
    x3 = x.reshape(B * C, H, W)
    zplane = jnp.zeros((H, W), x.dtype)

    mesh = plsc.VectorSubcoreMesh(core_axis_name="c", subcore_axis_name="s")
    NW = 32
    RPW = NP // NW

    @functools.partial(
        pl.kernel,
        out_type=jax.ShapeDtypeStruct((NP, H, W), x.dtype),
        mesh=mesh,
        scratch_types=[
            pltpu.VMEM((RPW,), jnp.int32),
            pltpu.VMEM((G, H, W), x.dtype),
            pltpu.VMEM((H, W), x.dtype),
            pltpu.SemaphoreType.DMA,
            pltpu.SemaphoreType.DMA,
        ],
        compiler_params=pltpu.CompilerParams(
            use_tc_tiling_on_sc=True, needs_layout_passes=False
        ),
    )
    def sc_gather(x_hbm, idx_hbm, z_hbm, out_hbm, idx_v, buf_v, zero_v, gsem, ssem):
        wid = lax.axis_index("s") * 2 + lax.axis_index("c")
        base = wid * RPW
        b = base // NF
        i0 = base % NF
        bC = b * C

        pltpu.sync_copy(idx_hbm.at[pl.ds(i0, RPW)], idx_v)
        pltpu.sync_copy(z_hbm, zero_v)

        def group(g, _):
            k0 = g * G
            iv = idx_v[pl.ds(k0, G)]
            nv = plsc.all_reduce_population_count(iv < C)[0]

            for j in range(G):
                v = iv[j]

                @pl.when(v < C)
                def _fire(j=j, v=v):
                    pltpu.async_copy(x_hbm.at[bC + v], buf_v.at[j], gsem)

            def drain(i, _):
                pltpu.make_async_copy(x_hbm.at[0], buf_v.at[0], gsem).wait()
                return 0

            lax.fori_loop(0, nv, drain, 0)

            for j in range(G):
                v = iv[j]

                @pl.when(v < C)
                def _store(j=j):
                    pltpu.async_copy(buf_v.at[j], out_hbm.at[base + k0 + j], ssem)

                @pl.when(v >= C)
                def _zero(j=j):
                    pltpu.async_copy(zero_v, out_hbm.at[base + k0 + j], ssem)

            def draw(i, _):
                pltpu.make_async_copy(zero_v, out_hbm.at[base], ssem).wait()
                return 0

            lax.fori_loop(0, G, draw, 0)
            return 0

        lax.fori_loop(0, RPW // G, group, 0)

    out = sc_gather(x3, indices, zplane)
    return out.reshape(B, NF, H, W)
